# Initial kernel scaffold; baseline (speedup 1.0000x reference)
import jax
import jax.numpy as jnp
from jax.experimental import pallas as pl


def kernel(maps, edge_row, tril_row, tril_col, left_idx, right_idx):
    raise NotImplementedError("write your pallas kernel here")


# speedup vs baseline: 5.1647x; 5.1647x over previous
import jax
import jax.numpy as jnp
from jax.experimental import pallas as pl

_N = 50000
_D = 4
_BLK = 6400


def _bmm_body(l_ref, r_ref, tt_ref, gl_ref, gr_ref):
    for i in range(4):
        for k in range(4):
            tl = l_ref[0 * 4 + i, :] * r_ref[0 * 4 + k, :]
            gl = l_ref[0 * 4 + i, :] * l_ref[0 * 4 + k, :]
            gr = r_ref[0 * 4 + i, :] * r_ref[0 * 4 + k, :]
            for j in range(1, 4):
                tl = tl + l_ref[j * 4 + i, :] * r_ref[j * 4 + k, :]
                gl = gl + l_ref[j * 4 + i, :] * l_ref[j * 4 + k, :]
                gr = gr + r_ref[j * 4 + i, :] * r_ref[j * 4 + k, :]
            tt_ref[i * 4 + k, :] = -tl
            gl_ref[i * 4 + k, :] = gl
            gr_ref[i * 4 + k, :] = gr


def _edge_products(mt, e):
    nb = e // _BLK
    return pl.pallas_call(
        _bmm_body,
        grid=(nb,),
        in_specs=[
            pl.BlockSpec((16, _BLK), lambda b: (0, b)),
            pl.BlockSpec((16, _BLK), lambda b, nb=nb: (0, b + nb)),
        ],
        out_specs=[
            pl.BlockSpec((16, _BLK), lambda b: (0, b)),
            pl.BlockSpec((16, _BLK), lambda b: (0, b)),
            pl.BlockSpec((16, _BLK), lambda b: (0, b)),
        ],
        out_shape=[
            jax.ShapeDtypeStruct((16, e), jnp.float32),
            jax.ShapeDtypeStruct((16, e), jnp.float32),
            jax.ShapeDtypeStruct((16, e), jnp.float32),
        ],
    )(mt, mt)


def kernel(maps, edge_row, tril_row, tril_col, left_idx, right_idx):
    e = tril_row.shape[0]
    mt = maps.reshape(2 * e, 16).T

    tt, gl, gr = _edge_products(mt, e)

    saved_tril_maps = tt.T.reshape(e, _D, _D)
    diag = jax.ops.segment_sum(gl.T, edge_row[:e], num_segments=_N)
    diag = diag + jax.ops.segment_sum(gr.T, edge_row[e:], num_segments=_N)

    tril_vals = tt.T.reshape(-1)
    diag_vals = diag.reshape(-1)

    ar = jnp.arange(_D, dtype=jnp.int32)
    tr = jnp.broadcast_to(
        tril_row[:, None, None] * _D + ar[None, :, None], (e, _D, _D)
    ).reshape(-1)
    tc = jnp.broadcast_to(
        tril_col[:, None, None] * _D + ar[None, None, :], (e, _D, _D)
    ).reshape(-1)
    nodes = jnp.arange(_N, dtype=jnp.int32)
    dr = jnp.broadcast_to(
        nodes[:, None, None] * _D + ar[None, :, None], (_N, _D, _D)
    ).reshape(-1)
    dc = jnp.broadcast_to(
        nodes[:, None, None] * _D + ar[None, None, :], (_N, _D, _D)
    ).reshape(-1)

    rows = jnp.concatenate([tr, tc, dr])
    cols = jnp.concatenate([tc, tr, dc])
    vals = jnp.concatenate([tril_vals, tril_vals, diag_vals])
    sort_key = rows * (_N * _D) + cols
    order = jnp.argsort(sort_key)
    out_index = jnp.stack([jnp.take(rows, order), jnp.take(cols, order)])
    out_weights = jnp.take(vals, order)
    return (out_index, out_weights), saved_tril_maps
